# Initial kernel scaffold; baseline (speedup 1.0000x reference)
import jax
import jax.numpy as jnp
from jax.experimental import pallas as pl


def kernel(embeddings, triplets):
    raise NotImplementedError("write your pallas kernel here")


# speedup vs baseline: 4.1281x; 4.1281x over previous
import jax
import jax.numpy as jnp
from jax import lax
from jax.experimental import pallas as pl
from jax.experimental.pallas import tpu as pltpu, tpu_sc as plsc

_MARGIN = 0.2
_EPS_INV = 1e12

_L = 16
_NC, _NS = 2, 16
_NW = _NC * _NS
_V, _D = 4096, 128
_T = 32768
_TPW = _T // _NW
_C = 64
_NCH = _TPW // _C
_RPW = _V // _NS


def _iota16():
    return lax.iota(jnp.int32, _L)


def _full16(v):
    return jnp.full((_L,), v, jnp.int32)


def _rsqrt16(x):
    i = plsc.bitcast(x, jnp.int32)
    i = jnp.int32(0x5F3759DF) - (i >> 1)
    y = plsc.bitcast(i, jnp.float32)
    for _ in range(3):
        y = y * (jnp.float32(1.5) - jnp.float32(0.5) * x * y * y)
    return jnp.minimum(y, jnp.float32(_EPS_INV))


def _colsum16(ref, g):
    rows = g * (_L * _L) + _iota16() * _L
    s = plsc.load_gather(ref, [rows])
    for l in range(1, _L):
        s = s + plsc.load_gather(ref, [rows + l])
    return s


def _sc_body(emb_hbm, trip_hbm, out_hbm,
             buf, part_ap, part_an, invn, tloc, aidx, pidx, nidx,
             accv, shared_inv, sem0, sem1):
    cid = lax.axis_index("c")
    sid = lax.axis_index("s")
    wid = sid * _NC + cid

    row0 = sid * _RPW
    for h in range(_RPW // _C):
        pltpu.sync_copy(emb_hbm.at[pl.ds(row0 + h * _C, _C)], buf.at[h])

        def sq_row(r, _):
            acc = jnp.zeros((_L,), jnp.float32)
            for j in range(_D // _L):
                v = buf[h, r, pl.ds(j * _L, _L)]
                acc = acc + v * v
            part_ap[pl.ds(r * _L, _L)] = acc
            return 0

        lax.fori_loop(0, _C, sq_row, 0)
        for g in range(_C // _L):
            ss = _colsum16(part_ap, g)
            invn[pl.ds(row0 + h * _C + g * _L, _L)] = _rsqrt16(ss)

    pltpu.sync_copy(invn.at[pl.ds(row0, _RPW)], shared_inv.at[pl.ds(row0, _RPW)])
    plsc.subcore_barrier()
    pltpu.sync_copy(shared_inv, invn)

    tbase = wid * _TPW
    pltpu.sync_copy(trip_hbm.at[pl.ds(tbase * 3, _TPW * 3)], tloc)

    def split_cols(g, _):
        rows3 = g * (_L * 3) + _iota16() * 3
        aidx[pl.ds(g * _L, _L)] = plsc.load_gather(tloc, [rows3])
        pidx[pl.ds(g * _L, _L)] = plsc.load_gather(tloc, [rows3 + 1])
        nidx[pl.ds(g * _L, _L)] = plsc.load_gather(tloc, [rows3 + 2])
        return 0

    lax.fori_loop(0, _TPW // _L, split_cols, 0)

    def _gather(c, slot, sem):
        for k, idx in enumerate((aidx, pidx, nidx)):
            pltpu.async_copy(emb_hbm.at[idx.at[pl.ds(c * _C, _C)]],
                             buf.at[3 * slot + k], sem)

    def _wait(c, slot, sem):
        for k, idx in enumerate((aidx, pidx, nidx)):
            pltpu.make_async_copy(emb_hbm.at[idx.at[pl.ds(c * _C, _C)]],
                                  buf.at[3 * slot + k], sem).wait()

    def _compute(c, slot, acc):
        a_ref = buf.at[3 * slot + 0]
        p_ref = buf.at[3 * slot + 1]
        n_ref = buf.at[3 * slot + 2]

        def dot_row(r, _):
            va0 = a_ref[r, pl.ds(0, _L)]
            ap = va0 * p_ref[r, pl.ds(0, _L)]
            an = va0 * n_ref[r, pl.ds(0, _L)]
            for j in range(1, _D // _L):
                va = a_ref[r, pl.ds(j * _L, _L)]
                ap = ap + va * p_ref[r, pl.ds(j * _L, _L)]
                an = an + va * n_ref[r, pl.ds(j * _L, _L)]
            part_ap[pl.ds(r * _L, _L)] = ap
            part_an[pl.ds(r * _L, _L)] = an
            return 0

        lax.fori_loop(0, _C, dot_row, 0)
        for g in range(_C // _L):
            sap = _colsum16(part_ap, g)
            san = _colsum16(part_an, g)
            base = c * _C + g * _L
            ia = plsc.load_gather(invn, [aidx[pl.ds(base, _L)]])
            ip = plsc.load_gather(invn, [pidx[pl.ds(base, _L)]])
            inn = plsc.load_gather(invn, [nidx[pl.ds(base, _L)]])
            d = jnp.float32(2.0) * ia * (san * inn - sap * ip)
            acc = acc + jnp.maximum(d + jnp.float32(_MARGIN), jnp.float32(0.0))
        return acc

    _gather(0, 0, sem0)
    _gather(1, 1, sem1)

    def chunk_pair(t, acc):
        c0 = 2 * t
        _wait(c0, 0, sem0)
        acc = _compute(c0, 0, acc)

        @pl.when(t < _NCH // 2 - 1)
        def _():
            _gather(c0 + 2, 0, sem0)

        _wait(c0 + 1, 1, sem1)
        acc = _compute(c0 + 1, 1, acc)

        @pl.when(t < _NCH // 2 - 1)
        def _():
            _gather(c0 + 3, 1, sem1)

        return acc

    acc = lax.fori_loop(0, _NCH // 2, chunk_pair, jnp.zeros((_L,), jnp.float32))
    accv[:] = acc
    pltpu.sync_copy(accv, out_hbm.at[wid])


def kernel(embeddings, triplets):
    emb = embeddings.astype(jnp.float32)
    trip = triplets.astype(jnp.int32).reshape(-1)

    mesh = plsc.VectorSubcoreMesh(core_axis_name="c", subcore_axis_name="s",
                                  num_cores=_NC, num_subcores=_NS)
    sc = pl.kernel(
        _sc_body,
        out_type=jax.ShapeDtypeStruct((_NW, _L), jnp.float32),
        mesh=mesh,
        compiler_params=pltpu.CompilerParams(needs_layout_passes=False),
        scratch_types=[
            pltpu.VMEM((6, _C, _D), jnp.float32),
            pltpu.VMEM((_C * _L,), jnp.float32),
            pltpu.VMEM((_C * _L,), jnp.float32),
            pltpu.VMEM((_V,), jnp.float32),
            pltpu.VMEM((_TPW * 3,), jnp.int32),
            pltpu.VMEM((_TPW,), jnp.int32),
            pltpu.VMEM((_TPW,), jnp.int32),
            pltpu.VMEM((_TPW,), jnp.int32),
            pltpu.VMEM((_L,), jnp.float32),
            pltpu.VMEM_SHARED((_V,), jnp.float32),
            pltpu.SemaphoreType.DMA,
            pltpu.SemaphoreType.DMA,
        ],
    )
    partial = sc(emb, trip)
    loss = jnp.sum(partial) / jnp.float32(_T)
    return (loss, triplets.shape[0])


# speedup vs baseline: 4.6332x; 1.1224x over previous
import jax
import jax.numpy as jnp
from jax import lax
from jax.experimental import pallas as pl
from jax.experimental.pallas import tpu as pltpu, tpu_sc as plsc

_MARGIN = 0.2
_EPS_INV = 1e12

_L = 16
_NC, _NS = 2, 16
_NW = _NC * _NS
_V, _D = 4096, 128
_T = 32768
_TPW = _T // _NW
_C = 64
_NCH = _TPW // _C
_RPW = _V // _NS


def _iota16():
    return lax.iota(jnp.int32, _L)


def _full16(v):
    return jnp.full((_L,), v, jnp.int32)


def _rsqrt16(x):
    i = plsc.bitcast(x, jnp.int32)
    i = jnp.int32(0x5F3759DF) - (i >> 1)
    y = plsc.bitcast(i, jnp.float32)
    for _ in range(3):
        y = y * (jnp.float32(1.5) - jnp.float32(0.5) * x * y * y)
    return jnp.minimum(y, jnp.float32(_EPS_INV))


def _colsum16(ref, g):
    rows = g * (_L * _L) + _iota16() * _L
    s = plsc.load_gather(ref, [rows])
    for l in range(1, _L):
        s = s + plsc.load_gather(ref, [rows + l])
    return s


def _sc_body(emb_hbm, trip_hbm, out_hbm,
             buf, part_ap, part_an, invn, ttile, aidx, pidx, nidx,
             accv, shared_inv, sem0, sem1, semt0, semt1):
    cid = lax.axis_index("c")
    sid = lax.axis_index("s")
    wid = sid * _NC + cid

    row0 = sid * _RPW
    for h in range(_RPW // _C):
        pltpu.sync_copy(emb_hbm.at[pl.ds(row0 + h * _C, _C)], buf.at[h])

        def sq_row(r, _):
            acc = jnp.zeros((_L,), jnp.float32)
            for j in range(_D // _L):
                v = buf[h, r, pl.ds(j * _L, _L)]
                acc = acc + v * v
            part_ap[pl.ds(r * _L, _L)] = acc
            return 0

        lax.fori_loop(0, _C, sq_row, 0)
        for g in range(_C // _L):
            ss = _colsum16(part_ap, g)
            invn[pl.ds(row0 + h * _C + g * _L, _L)] = _rsqrt16(ss)

    pltpu.sync_copy(invn.at[pl.ds(row0, _RPW)], shared_inv.at[pl.ds(row0, _RPW)])
    plsc.subcore_barrier()
    pltpu.sync_copy(shared_inv, invn)

    tbase = wid * _TPW

    def _trip_dma(c, slot, sem):
        pltpu.async_copy(trip_hbm.at[pl.ds(tbase + c * _C, _C)],
                         ttile.at[slot], sem)

    def _trip_wait(c, slot, sem):
        pltpu.make_async_copy(trip_hbm.at[pl.ds(tbase + c * _C, _C)],
                              ttile.at[slot], sem).wait()

    def _extract(slot):
        tt = ttile.at[slot]
        for g in range(_C // _L):
            rows = g * _L + _iota16()
            dst = pl.ds(slot * _C + g * _L, _L)
            aidx[dst] = plsc.load_gather(tt, [rows, _full16(0)])
            pidx[dst] = plsc.load_gather(tt, [rows, _full16(1)])
            nidx[dst] = plsc.load_gather(tt, [rows, _full16(2)])

    def _gather(slot, sem):
        for k, idx in enumerate((aidx, pidx, nidx)):
            pltpu.async_copy(emb_hbm.at[idx.at[pl.ds(slot * _C, _C)]],
                             buf.at[3 * slot + k], sem)

    def _gather_wait(slot, sem):
        for k, idx in enumerate((aidx, pidx, nidx)):
            pltpu.make_async_copy(emb_hbm.at[idx.at[pl.ds(slot * _C, _C)]],
                                  buf.at[3 * slot + k], sem).wait()

    def _compute(slot, acc):
        a_ref = buf.at[3 * slot + 0]
        p_ref = buf.at[3 * slot + 1]
        n_ref = buf.at[3 * slot + 2]

        def dot2_row(i, _):
            for u in range(2):
                r = 2 * i + u
                va0 = a_ref[r, pl.ds(0, _L)]
                ap = va0 * p_ref[r, pl.ds(0, _L)]
                an = va0 * n_ref[r, pl.ds(0, _L)]
                for j in range(1, _D // _L):
                    va = a_ref[r, pl.ds(j * _L, _L)]
                    ap = ap + va * p_ref[r, pl.ds(j * _L, _L)]
                    an = an + va * n_ref[r, pl.ds(j * _L, _L)]
                part_ap[pl.ds(r * _L, _L)] = ap
                part_an[pl.ds(r * _L, _L)] = an
            return 0

        lax.fori_loop(0, _C // 2, dot2_row, 0)
        for g in range(_C // _L):
            sap = _colsum16(part_ap, g)
            san = _colsum16(part_an, g)
            src = pl.ds(slot * _C + g * _L, _L)
            ia = plsc.load_gather(invn, [aidx[src]])
            ip = plsc.load_gather(invn, [pidx[src]])
            inn = plsc.load_gather(invn, [nidx[src]])
            d = jnp.float32(2.0) * ia * (san * inn - sap * ip)
            acc = acc + jnp.maximum(d + jnp.float32(_MARGIN), jnp.float32(0.0))
        return acc

    _trip_dma(0, 0, semt0)
    _trip_dma(1, 1, semt1)
    _trip_wait(0, 0, semt0)
    _extract(0)
    _gather(0, sem0)
    _trip_wait(1, 1, semt1)
    _extract(1)
    _gather(1, sem1)

    def chunk_pair(t, acc):
        c0 = 2 * t
        more = t < _NCH // 2 - 1

        @pl.when(more)
        def _():
            _trip_dma(c0 + 2, 0, semt0)

        _gather_wait(0, sem0)
        acc = _compute(0, acc)

        @pl.when(more)
        def _():
            _trip_wait(c0 + 2, 0, semt0)
            _extract(0)
            _gather(0, sem0)
            _trip_dma(c0 + 3, 1, semt1)

        _gather_wait(1, sem1)
        acc = _compute(1, acc)

        @pl.when(more)
        def _():
            _trip_wait(c0 + 3, 1, semt1)
            _extract(1)
            _gather(1, sem1)

        return acc

    acc = lax.fori_loop(0, _NCH // 2, chunk_pair, jnp.zeros((_L,), jnp.float32))
    accv[:] = acc
    pltpu.sync_copy(accv, out_hbm.at[wid])


def kernel(embeddings, triplets):
    emb = embeddings.astype(jnp.float32)
    trip = triplets.astype(jnp.int32)

    mesh = plsc.VectorSubcoreMesh(core_axis_name="c", subcore_axis_name="s",
                                  num_cores=_NC, num_subcores=_NS)
    sc = pl.kernel(
        _sc_body,
        out_type=jax.ShapeDtypeStruct((_NW, _L), jnp.float32),
        mesh=mesh,
        compiler_params=pltpu.CompilerParams(needs_layout_passes=False),
        scratch_types=[
            pltpu.VMEM((6, _C, _D), jnp.float32),
            pltpu.VMEM((_C * _L,), jnp.float32),
            pltpu.VMEM((_C * _L,), jnp.float32),
            pltpu.VMEM((_V,), jnp.float32),
            pltpu.VMEM((2, _C, 3), jnp.int32),
            pltpu.VMEM((2 * _C,), jnp.int32),
            pltpu.VMEM((2 * _C,), jnp.int32),
            pltpu.VMEM((2 * _C,), jnp.int32),
            pltpu.VMEM((_L,), jnp.float32),
            pltpu.VMEM_SHARED((_V,), jnp.float32),
            pltpu.SemaphoreType.DMA,
            pltpu.SemaphoreType.DMA,
            pltpu.SemaphoreType.DMA,
            pltpu.SemaphoreType.DMA,
        ],
    )
    partial = sc(emb, trip)
    loss = jnp.sum(partial) / jnp.float32(_T)
    return (loss, triplets.shape[0])


# speedup vs baseline: 4.6576x; 1.0053x over previous
import jax
import jax.numpy as jnp
from jax import lax
from jax.experimental import pallas as pl
from jax.experimental.pallas import tpu as pltpu, tpu_sc as plsc

_MARGIN = 0.2
_EPS_INV = 1e12

_L = 16
_NC, _NS = 2, 16
_NW = _NC * _NS
_V, _D = 4096, 128
_T = 32768
_TPW = _T // _NW
_C = 64
_NCH = _TPW // _C
_RPW = _V // _NS


def _iota16():
    return lax.iota(jnp.int32, _L)


def _full16(v):
    return jnp.full((_L,), v, jnp.int32)


def _rsqrt16(x):
    i = plsc.bitcast(x, jnp.int32)
    i = jnp.int32(0x5F3759DF) - (i >> 1)
    y = plsc.bitcast(i, jnp.float32)
    for _ in range(3):
        y = y * (jnp.float32(1.5) - jnp.float32(0.5) * x * y * y)
    return jnp.minimum(y, jnp.float32(_EPS_INV))


def _colsum16(ref, g):
    rows = g * (_L * _L) + _iota16() * _L
    s = plsc.load_gather(ref, [rows])
    for l in range(1, _L):
        s = s + plsc.load_gather(ref, [rows + l])
    return s


def _sc_body(emb_hbm, trip_hbm, out_hbm,
             buf, part_ap, part_an, invn, ttile, aidx, pidx, nidx,
             accv, shared_inv, sem0, sem1, semt0, semt1):
    cid = lax.axis_index("c")
    sid = lax.axis_index("s")
    wid = sid * _NC + cid

    row0 = sid * _RPW
    for h in range(_RPW // _C):
        pltpu.sync_copy(emb_hbm.at[pl.ds(row0 + h * _C, _C)], buf.at[h])

        def sq_row(r, _):
            s = [jnp.zeros((_L,), jnp.float32)] * 4
            for j in range(_D // _L):
                v = buf[h, r, pl.ds(j * _L, _L)]
                s[j % 4] = s[j % 4] + v * v
            part_ap[pl.ds(r * _L, _L)] = (s[0] + s[1]) + (s[2] + s[3])
            return 0

        lax.fori_loop(0, _C, sq_row, 0)
        for g in range(_C // _L):
            ss = _colsum16(part_ap, g)
            invn[pl.ds(row0 + h * _C + g * _L, _L)] = _rsqrt16(ss)

    pltpu.sync_copy(invn.at[pl.ds(row0, _RPW)], shared_inv.at[pl.ds(row0, _RPW)])
    plsc.subcore_barrier()
    pltpu.sync_copy(shared_inv, invn)

    tbase = wid * _TPW

    def _trip_dma(c, slot, sem):
        pltpu.async_copy(trip_hbm.at[pl.ds(tbase + c * _C, _C)],
                         ttile.at[slot], sem)

    def _trip_wait(c, slot, sem):
        pltpu.make_async_copy(trip_hbm.at[pl.ds(tbase + c * _C, _C)],
                              ttile.at[slot], sem).wait()

    def _extract(slot):
        tt = ttile.at[slot]
        for g in range(_C // _L):
            rows = g * _L + _iota16()
            dst = pl.ds(slot * _C + g * _L, _L)
            aidx[dst] = plsc.load_gather(tt, [rows, _full16(0)])
            pidx[dst] = plsc.load_gather(tt, [rows, _full16(1)])
            nidx[dst] = plsc.load_gather(tt, [rows, _full16(2)])

    def _gather(slot, sem):
        for k, idx in enumerate((aidx, pidx, nidx)):
            pltpu.async_copy(emb_hbm.at[idx.at[pl.ds(slot * _C, _C)]],
                             buf.at[3 * slot + k], sem)

    def _gather_wait(slot, sem):
        for k, idx in enumerate((aidx, pidx, nidx)):
            pltpu.make_async_copy(emb_hbm.at[idx.at[pl.ds(slot * _C, _C)]],
                                  buf.at[3 * slot + k], sem).wait()

    def _compute(slot, acc):
        a_ref = buf.at[3 * slot + 0]
        p_ref = buf.at[3 * slot + 1]
        n_ref = buf.at[3 * slot + 2]

        def dot2_row(i, _):
            for u in range(2):
                r = 2 * i + u
                ap0 = jnp.zeros((_L,), jnp.float32)
                ap1 = jnp.zeros((_L,), jnp.float32)
                an0 = jnp.zeros((_L,), jnp.float32)
                an1 = jnp.zeros((_L,), jnp.float32)
                for j in range(0, _D // _L, 2):
                    va = a_ref[r, pl.ds(j * _L, _L)]
                    vb = a_ref[r, pl.ds((j + 1) * _L, _L)]
                    ap0 = ap0 + va * p_ref[r, pl.ds(j * _L, _L)]
                    ap1 = ap1 + vb * p_ref[r, pl.ds((j + 1) * _L, _L)]
                    an0 = an0 + va * n_ref[r, pl.ds(j * _L, _L)]
                    an1 = an1 + vb * n_ref[r, pl.ds((j + 1) * _L, _L)]
                part_ap[pl.ds(r * _L, _L)] = ap0 + ap1
                part_an[pl.ds(r * _L, _L)] = an0 + an1
            return 0

        lax.fori_loop(0, _C // 2, dot2_row, 0)
        for g in range(_C // _L):
            sap = _colsum16(part_ap, g)
            san = _colsum16(part_an, g)
            src = pl.ds(slot * _C + g * _L, _L)
            ia = plsc.load_gather(invn, [aidx[src]])
            ip = plsc.load_gather(invn, [pidx[src]])
            inn = plsc.load_gather(invn, [nidx[src]])
            d = jnp.float32(2.0) * ia * (san * inn - sap * ip)
            acc = acc + jnp.maximum(d + jnp.float32(_MARGIN), jnp.float32(0.0))
        return acc

    _trip_dma(0, 0, semt0)
    _trip_dma(1, 1, semt1)
    _trip_wait(0, 0, semt0)
    _extract(0)
    _gather(0, sem0)
    _trip_wait(1, 1, semt1)
    _extract(1)
    _gather(1, sem1)

    def chunk_pair(t, acc):
        c0 = 2 * t
        more = t < _NCH // 2 - 1

        @pl.when(more)
        def _():
            _trip_dma(c0 + 2, 0, semt0)

        _gather_wait(0, sem0)
        acc = _compute(0, acc)

        @pl.when(more)
        def _():
            _trip_wait(c0 + 2, 0, semt0)
            _extract(0)
            _gather(0, sem0)
            _trip_dma(c0 + 3, 1, semt1)

        _gather_wait(1, sem1)
        acc = _compute(1, acc)

        @pl.when(more)
        def _():
            _trip_wait(c0 + 3, 1, semt1)
            _extract(1)
            _gather(1, sem1)

        return acc

    acc = lax.fori_loop(0, _NCH // 2, chunk_pair, jnp.zeros((_L,), jnp.float32))
    accv[:] = acc
    pltpu.sync_copy(accv, out_hbm.at[wid])


def kernel(embeddings, triplets):
    emb = embeddings.astype(jnp.float32)
    trip = triplets.astype(jnp.int32)

    mesh = plsc.VectorSubcoreMesh(core_axis_name="c", subcore_axis_name="s",
                                  num_cores=_NC, num_subcores=_NS)
    sc = pl.kernel(
        _sc_body,
        out_type=jax.ShapeDtypeStruct((_NW, _L), jnp.float32),
        mesh=mesh,
        compiler_params=pltpu.CompilerParams(needs_layout_passes=False),
        scratch_types=[
            pltpu.VMEM((6, _C, _D), jnp.float32),
            pltpu.VMEM((_C * _L,), jnp.float32),
            pltpu.VMEM((_C * _L,), jnp.float32),
            pltpu.VMEM((_V,), jnp.float32),
            pltpu.VMEM((2, _C, 3), jnp.int32),
            pltpu.VMEM((2 * _C,), jnp.int32),
            pltpu.VMEM((2 * _C,), jnp.int32),
            pltpu.VMEM((2 * _C,), jnp.int32),
            pltpu.VMEM((_L,), jnp.float32),
            pltpu.VMEM_SHARED((_V,), jnp.float32),
            pltpu.SemaphoreType.DMA,
            pltpu.SemaphoreType.DMA,
            pltpu.SemaphoreType.DMA,
            pltpu.SemaphoreType.DMA,
        ],
    )
    partial = sc(emb, trip)
    loss = jnp.sum(partial) / jnp.float32(_T)
    return (loss, triplets.shape[0])


# speedup vs baseline: 4.9359x; 1.0597x over previous
import jax
import jax.numpy as jnp
from jax import lax
from jax.experimental import pallas as pl
from jax.experimental.pallas import tpu as pltpu, tpu_sc as plsc

_MARGIN = 0.2
_EPS_INV = 1e12

_L = 16
_NC, _NS = 2, 16
_NW = _NC * _NS
_V, _D = 4096, 128
_T = 32768
_TPW = _T // _NW
_C = 64
_NCH = _TPW // _C
_RPW = _V // _NS


def _iota16():
    return lax.iota(jnp.int32, _L)


def _full16(v):
    return jnp.full((_L,), v, jnp.int32)


def _rsqrt16(x):
    i = plsc.bitcast(x, jnp.int32)
    i = jnp.int32(0x5F3759DF) - (i >> 1)
    y = plsc.bitcast(i, jnp.float32)
    for _ in range(3):
        y = y * (jnp.float32(1.5) - jnp.float32(0.5) * x * y * y)
    return jnp.minimum(y, jnp.float32(_EPS_INV))


def _colsum16(ref, g):
    rows = g * (_L * _L) + _iota16() * _L
    s = plsc.load_gather(ref, [rows])
    for l in range(1, _L):
        s = s + plsc.load_gather(ref, [rows + l])
    return s


def _sc_body(emb_hbm, trip_hbm, out_hbm,
             buf, part_ap, part_an, invn, ttile, aidx, pidx, nidx,
             accv, shared_inv, sem0, sem1, semt0, semt1):
    cid = lax.axis_index("c")
    sid = lax.axis_index("s")
    wid = sid * _NC + cid

    row0 = sid * _RPW
    for h in range(_RPW // _C):
        pltpu.sync_copy(emb_hbm.at[pl.ds(row0 + h * _C, _C)], buf.at[h])

        @plsc.parallel_loop(0, _C, unroll=2)
        def sq_row(r):
            s = [jnp.zeros((_L,), jnp.float32)] * 4
            for j in range(_D // _L):
                v = buf[h, r, pl.ds(j * _L, _L)]
                s[j % 4] = s[j % 4] + v * v
            part_ap[pl.ds(r * _L, _L)] = (s[0] + s[1]) + (s[2] + s[3])
        for g in range(_C // _L):
            ss = _colsum16(part_ap, g)
            invn[pl.ds(row0 + h * _C + g * _L, _L)] = _rsqrt16(ss)

    pltpu.sync_copy(invn.at[pl.ds(row0, _RPW)], shared_inv.at[pl.ds(row0, _RPW)])
    plsc.subcore_barrier()
    pltpu.sync_copy(shared_inv, invn)

    tbase = wid * _TPW

    def _trip_dma(c, slot, sem):
        pltpu.async_copy(trip_hbm.at[pl.ds(tbase + c * _C, _C)],
                         ttile.at[slot], sem)

    def _trip_wait(c, slot, sem):
        pltpu.make_async_copy(trip_hbm.at[pl.ds(tbase + c * _C, _C)],
                              ttile.at[slot], sem).wait()

    def _extract(slot):
        tt = ttile.at[slot]
        for g in range(_C // _L):
            rows = g * _L + _iota16()
            dst = pl.ds(slot * _C + g * _L, _L)
            aidx[dst] = plsc.load_gather(tt, [rows, _full16(0)])
            pidx[dst] = plsc.load_gather(tt, [rows, _full16(1)])
            nidx[dst] = plsc.load_gather(tt, [rows, _full16(2)])

    def _gather(slot, sem):
        for k, idx in enumerate((aidx, pidx, nidx)):
            pltpu.async_copy(emb_hbm.at[idx.at[pl.ds(slot * _C, _C)]],
                             buf.at[3 * slot + k], sem)

    def _gather_wait(slot, sem):
        for k, idx in enumerate((aidx, pidx, nidx)):
            pltpu.make_async_copy(emb_hbm.at[idx.at[pl.ds(slot * _C, _C)]],
                                  buf.at[3 * slot + k], sem).wait()

    def _compute(slot, acc):
        a_ref = buf.at[3 * slot + 0]
        p_ref = buf.at[3 * slot + 1]
        n_ref = buf.at[3 * slot + 2]

        @plsc.parallel_loop(0, _C, unroll=2)
        def dot_row(r):
            ap0 = jnp.zeros((_L,), jnp.float32)
            ap1 = jnp.zeros((_L,), jnp.float32)
            an0 = jnp.zeros((_L,), jnp.float32)
            an1 = jnp.zeros((_L,), jnp.float32)
            for j in range(0, _D // _L, 2):
                va = a_ref[r, pl.ds(j * _L, _L)]
                vb = a_ref[r, pl.ds((j + 1) * _L, _L)]
                ap0 = ap0 + va * p_ref[r, pl.ds(j * _L, _L)]
                ap1 = ap1 + vb * p_ref[r, pl.ds((j + 1) * _L, _L)]
                an0 = an0 + va * n_ref[r, pl.ds(j * _L, _L)]
                an1 = an1 + vb * n_ref[r, pl.ds((j + 1) * _L, _L)]
            part_ap[pl.ds(r * _L, _L)] = ap0 + ap1
            part_an[pl.ds(r * _L, _L)] = an0 + an1
        for g in range(_C // _L):
            sap = _colsum16(part_ap, g)
            san = _colsum16(part_an, g)
            src = pl.ds(slot * _C + g * _L, _L)
            ia = plsc.load_gather(invn, [aidx[src]])
            ip = plsc.load_gather(invn, [pidx[src]])
            inn = plsc.load_gather(invn, [nidx[src]])
            d = jnp.float32(2.0) * ia * (san * inn - sap * ip)
            acc = acc + jnp.maximum(d + jnp.float32(_MARGIN), jnp.float32(0.0))
        return acc

    _trip_dma(0, 0, semt0)
    _trip_dma(1, 1, semt1)
    _trip_wait(0, 0, semt0)
    _extract(0)
    _gather(0, sem0)
    _trip_wait(1, 1, semt1)
    _extract(1)
    _gather(1, sem1)

    def chunk_pair(t, acc):
        c0 = 2 * t
        more = t < _NCH // 2 - 1

        @pl.when(more)
        def _():
            _trip_dma(c0 + 2, 0, semt0)

        _gather_wait(0, sem0)
        acc = _compute(0, acc)

        @pl.when(more)
        def _():
            _trip_wait(c0 + 2, 0, semt0)
            _extract(0)
            _gather(0, sem0)
            _trip_dma(c0 + 3, 1, semt1)

        _gather_wait(1, sem1)
        acc = _compute(1, acc)

        @pl.when(more)
        def _():
            _trip_wait(c0 + 3, 1, semt1)
            _extract(1)
            _gather(1, sem1)

        return acc

    acc = lax.fori_loop(0, _NCH // 2, chunk_pair, jnp.zeros((_L,), jnp.float32))
    accv[:] = acc
    pltpu.sync_copy(accv, out_hbm.at[wid])


def kernel(embeddings, triplets):
    emb = embeddings.astype(jnp.float32)
    trip = triplets.astype(jnp.int32)

    mesh = plsc.VectorSubcoreMesh(core_axis_name="c", subcore_axis_name="s",
                                  num_cores=_NC, num_subcores=_NS)
    sc = pl.kernel(
        _sc_body,
        out_type=jax.ShapeDtypeStruct((_NW, _L), jnp.float32),
        mesh=mesh,
        compiler_params=pltpu.CompilerParams(needs_layout_passes=False),
        scratch_types=[
            pltpu.VMEM((6, _C, _D), jnp.float32),
            pltpu.VMEM((_C * _L,), jnp.float32),
            pltpu.VMEM((_C * _L,), jnp.float32),
            pltpu.VMEM((_V,), jnp.float32),
            pltpu.VMEM((2, _C, 3), jnp.int32),
            pltpu.VMEM((2 * _C,), jnp.int32),
            pltpu.VMEM((2 * _C,), jnp.int32),
            pltpu.VMEM((2 * _C,), jnp.int32),
            pltpu.VMEM((_L,), jnp.float32),
            pltpu.VMEM_SHARED((_V,), jnp.float32),
            pltpu.SemaphoreType.DMA,
            pltpu.SemaphoreType.DMA,
            pltpu.SemaphoreType.DMA,
            pltpu.SemaphoreType.DMA,
        ],
    )
    partial = sc(emb, trip)
    loss = jnp.sum(partial) / jnp.float32(_T)
    return (loss, triplets.shape[0])


# speedup vs baseline: 4.9500x; 1.0029x over previous
import jax
import jax.numpy as jnp
from jax import lax
from jax.experimental import pallas as pl
from jax.experimental.pallas import tpu as pltpu, tpu_sc as plsc

_MARGIN = 0.2
_EPS_INV = 1e12

_L = 16
_NC, _NS = 2, 16
_NW = _NC * _NS
_V, _D = 4096, 128
_T = 32768
_TPW = _T // _NW
_C = 64
_NCH = _TPW // _C
_RPW = _V // _NS


def _iota16():
    return lax.iota(jnp.int32, _L)


def _full16(v):
    return jnp.full((_L,), v, jnp.int32)


def _rsqrt16(x):
    i = plsc.bitcast(x, jnp.int32)
    i = jnp.int32(0x5F3759DF) - (i >> 1)
    y = plsc.bitcast(i, jnp.float32)
    for _ in range(3):
        y = y * (jnp.float32(1.5) - jnp.float32(0.5) * x * y * y)
    return jnp.minimum(y, jnp.float32(_EPS_INV))


def _colsum16(ref, g):
    rows = g * (_L * _L) + _iota16() * _L
    s = plsc.load_gather(ref, [rows])
    for l in range(1, _L):
        s = s + plsc.load_gather(ref, [rows + l])
    return s


def _sc_body(emb_hbm, trip_hbm, out_hbm,
             buf, part_ap, part_an, invn, ttile, aidx, pidx, nidx,
             accv, shared_inv, sem0, sem1, semt0, semt1):
    cid = lax.axis_index("c")
    sid = lax.axis_index("s")
    wid = sid * _NC + cid

    row0 = sid * _RPW
    for h in range(_RPW // _C):
        pltpu.sync_copy(emb_hbm.at[pl.ds(row0 + h * _C, _C)], buf.at[h])

        @plsc.parallel_loop(0, _C, unroll=2)
        def sq_row(r):
            s = [jnp.zeros((_L,), jnp.float32)] * 4
            for j in range(_D // _L):
                v = buf[h, r, pl.ds(j * _L, _L)]
                s[j % 4] = s[j % 4] + v * v
            part_ap[pl.ds(r * _L, _L)] = (s[0] + s[1]) + (s[2] + s[3])
        for g in range(_C // _L):
            ss = _colsum16(part_ap, g)
            invn[pl.ds(row0 + h * _C + g * _L, _L)] = _rsqrt16(ss)

    pltpu.sync_copy(invn.at[pl.ds(row0, _RPW)], shared_inv.at[pl.ds(row0, _RPW)])
    plsc.subcore_barrier()
    pltpu.sync_copy(shared_inv, invn)

    tbase = wid * _TPW

    def _trip_dma(c, slot, sem):
        pltpu.async_copy(trip_hbm.at[pl.ds(tbase + c * _C, _C)],
                         ttile.at[slot], sem)

    def _trip_wait(c, slot, sem):
        pltpu.make_async_copy(trip_hbm.at[pl.ds(tbase + c * _C, _C)],
                              ttile.at[slot], sem).wait()

    def _extract(slot):
        tt = ttile.at[slot]
        for g in range(_C // _L):
            rows = g * _L + _iota16()
            dst = pl.ds(slot * _C + g * _L, _L)
            aidx[dst] = plsc.load_gather(tt, [rows, _full16(0)])
            pidx[dst] = plsc.load_gather(tt, [rows, _full16(1)])
            nidx[dst] = plsc.load_gather(tt, [rows, _full16(2)])

    def _gather(slot, sem):
        for k, idx in enumerate((aidx, pidx, nidx)):
            pltpu.async_copy(emb_hbm.at[idx.at[pl.ds(slot * _C, _C)]],
                             buf.at[3 * slot + k], sem)

    def _gather_wait(slot, sem):
        for k, idx in enumerate((aidx, pidx, nidx)):
            pltpu.make_async_copy(emb_hbm.at[idx.at[pl.ds(slot * _C, _C)]],
                                  buf.at[3 * slot + k], sem).wait()

    def _compute(slot, acc):
        a_ref = buf.at[3 * slot + 0]
        p_ref = buf.at[3 * slot + 1]
        n_ref = buf.at[3 * slot + 2]

        @plsc.parallel_loop(0, _C, unroll=2)
        def dot_row(r):
            ap0 = jnp.zeros((_L,), jnp.float32)
            ap1 = jnp.zeros((_L,), jnp.float32)
            an0 = jnp.zeros((_L,), jnp.float32)
            an1 = jnp.zeros((_L,), jnp.float32)
            for j in range(0, _D // _L, 2):
                va = a_ref[r, pl.ds(j * _L, _L)]
                vb = a_ref[r, pl.ds((j + 1) * _L, _L)]
                ap0 = ap0 + va * p_ref[r, pl.ds(j * _L, _L)]
                ap1 = ap1 + vb * p_ref[r, pl.ds((j + 1) * _L, _L)]
                an0 = an0 + va * n_ref[r, pl.ds(j * _L, _L)]
                an1 = an1 + vb * n_ref[r, pl.ds((j + 1) * _L, _L)]
            part_ap[pl.ds(r * _L, _L)] = ap0 + ap1
            part_an[pl.ds(r * _L, _L)] = an0 + an1
        for g in range(_C // _L):
            sap = _colsum16(part_ap, g)
            san = _colsum16(part_an, g)
            src = pl.ds(slot * _C + g * _L, _L)
            ia = plsc.load_gather(invn, [aidx[src]])
            ip = plsc.load_gather(invn, [pidx[src]])
            inn = plsc.load_gather(invn, [nidx[src]])
            d = jnp.float32(2.0) * ia * (san * inn - sap * ip)
            acc = acc + jnp.maximum(d + jnp.float32(_MARGIN), jnp.float32(0.0))
        return acc

    _trip_dma(0, 0, semt0)
    _trip_dma(1, 1, semt1)
    _trip_wait(0, 0, semt0)
    _extract(0)
    _gather(0, sem0)
    _trip_wait(1, 1, semt1)
    _extract(1)
    _gather(1, sem1)

    def chunk_pair(t, acc):
        c0 = 2 * t
        more = t < _NCH // 2 - 1

        @pl.when(more)
        def _():
            _trip_dma(c0 + 2, 0, semt0)

        _gather_wait(0, sem0)
        acc = _compute(0, acc)

        @pl.when(more)
        def _():
            _trip_wait(c0 + 2, 0, semt0)
            _extract(0)
            _gather(0, sem0)
            _trip_dma(c0 + 3, 1, semt1)

        _gather_wait(1, sem1)
        acc = _compute(1, acc)

        @pl.when(more)
        def _():
            _trip_wait(c0 + 3, 1, semt1)
            _extract(1)
            _gather(1, sem1)

        return acc

    acc = lax.fori_loop(0, _NCH // 2, chunk_pair, jnp.zeros((_L,), jnp.float32))
    accv[:] = acc
    pltpu.sync_copy(accv, out_hbm.at[wid])


def kernel(embeddings, triplets):
    emb = embeddings.astype(jnp.float32)
    trip = triplets.astype(jnp.int32)

    mesh = plsc.VectorSubcoreMesh(core_axis_name="c", subcore_axis_name="s",
                                  num_cores=_NC, num_subcores=_NS)
    sc = pl.kernel(
        _sc_body,
        out_type=jax.ShapeDtypeStruct((_NW, _L), jnp.float32),
        mesh=mesh,
        compiler_params=pltpu.CompilerParams(needs_layout_passes=False,
                                             use_tc_tiling_on_sc=True),
        scratch_types=[
            pltpu.VMEM((6, _C, _D), jnp.float32),
            pltpu.VMEM((_C * _L,), jnp.float32),
            pltpu.VMEM((_C * _L,), jnp.float32),
            pltpu.VMEM((_V,), jnp.float32),
            pltpu.VMEM((2, _C, 3), jnp.int32),
            pltpu.VMEM((2 * _C,), jnp.int32),
            pltpu.VMEM((2 * _C,), jnp.int32),
            pltpu.VMEM((2 * _C,), jnp.int32),
            pltpu.VMEM((_L,), jnp.float32),
            pltpu.VMEM_SHARED((_V,), jnp.float32),
            pltpu.SemaphoreType.DMA,
            pltpu.SemaphoreType.DMA,
            pltpu.SemaphoreType.DMA,
            pltpu.SemaphoreType.DMA,
        ],
    )
    partial = sc(emb, trip)
    loss = jnp.sum(partial) / jnp.float32(_T)
    return (loss, triplets.shape[0])


# speedup vs baseline: 5.5706x; 1.1254x over previous
import jax
import jax.numpy as jnp
from jax import lax
from jax.experimental import pallas as pl
from jax.experimental.pallas import tpu as pltpu, tpu_sc as plsc

_MARGIN = 0.2
_EPS_INV = 1e12

_L = 16
_NC, _NS = 2, 16
_NW = _NC * _NS
_V, _D = 4096, 128
_T = 32768
_TPW = _T // _NW
_C = 64
_NCH = _TPW // _C
_RPW = _V // _NS


def _iota16():
    return lax.iota(jnp.int32, _L)


def _full16(v):
    return jnp.full((_L,), v, jnp.int32)


def _rsqrt16(x):
    i = plsc.bitcast(x, jnp.int32)
    i = jnp.int32(0x5F3759DF) - (i >> 1)
    y = plsc.bitcast(i, jnp.float32)
    for _ in range(3):
        y = y * (jnp.float32(1.5) - jnp.float32(0.5) * x * y * y)
    return jnp.minimum(y, jnp.float32(_EPS_INV))


def _colsum16(ref, g):
    rows = g * (_L * _L) + _iota16() * _L
    s = plsc.load_gather(ref, [rows])
    for l in range(1, _L):
        s = s + plsc.load_gather(ref, [rows + l])
    return s


def _sc_body(emb_hbm, ai_hbm, pi_hbm, ni_hbm, out_hbm,
             buf, part_ap, part_an, invn, aidx, pidx, nidx,
             accv, shared_inv, sem0, sem1):
    cid = lax.axis_index("c")
    sid = lax.axis_index("s")
    wid = sid * _NC + cid

    row0 = sid * _RPW
    for h in range(_RPW // _C):
        pltpu.sync_copy(emb_hbm.at[pl.ds(row0 + h * _C, _C)], buf.at[h])

        @plsc.parallel_loop(0, _C, unroll=2)
        def sq_row(r):
            s = [jnp.zeros((_L,), jnp.float32)] * 4
            for j in range(_D // _L):
                v = buf[h, r, pl.ds(j * _L, _L)]
                s[j % 4] = s[j % 4] + v * v
            part_ap[pl.ds(r * _L, _L)] = (s[0] + s[1]) + (s[2] + s[3])
        for g in range(_C // _L):
            ss = _colsum16(part_ap, g)
            invn[pl.ds(row0 + h * _C + g * _L, _L)] = _rsqrt16(ss)

    pltpu.sync_copy(invn.at[pl.ds(row0, _RPW)], shared_inv.at[pl.ds(row0, _RPW)])
    plsc.subcore_barrier()
    pltpu.sync_copy(shared_inv, invn)

    tbase = wid * _TPW
    pltpu.sync_copy(ai_hbm.at[pl.ds(tbase, _TPW)], aidx)
    pltpu.sync_copy(pi_hbm.at[pl.ds(tbase, _TPW)], pidx)
    pltpu.sync_copy(ni_hbm.at[pl.ds(tbase, _TPW)], nidx)

    def _gather(c, slot, sem):
        for k, idx in enumerate((aidx, pidx, nidx)):
            pltpu.async_copy(emb_hbm.at[idx.at[pl.ds(c * _C, _C)]],
                             buf.at[3 * slot + k], sem)

    def _gather_wait(c, slot, sem):
        for k, idx in enumerate((aidx, pidx, nidx)):
            pltpu.make_async_copy(emb_hbm.at[idx.at[pl.ds(c * _C, _C)]],
                                  buf.at[3 * slot + k], sem).wait()

    def _compute(c, slot, acc):
        a_ref = buf.at[3 * slot + 0]
        p_ref = buf.at[3 * slot + 1]
        n_ref = buf.at[3 * slot + 2]

        @plsc.parallel_loop(0, _C, unroll=2)
        def dot_row(r):
            ap0 = jnp.zeros((_L,), jnp.float32)
            ap1 = jnp.zeros((_L,), jnp.float32)
            an0 = jnp.zeros((_L,), jnp.float32)
            an1 = jnp.zeros((_L,), jnp.float32)
            for j in range(0, _D // _L, 2):
                va = a_ref[r, pl.ds(j * _L, _L)]
                vb = a_ref[r, pl.ds((j + 1) * _L, _L)]
                ap0 = ap0 + va * p_ref[r, pl.ds(j * _L, _L)]
                ap1 = ap1 + vb * p_ref[r, pl.ds((j + 1) * _L, _L)]
                an0 = an0 + va * n_ref[r, pl.ds(j * _L, _L)]
                an1 = an1 + vb * n_ref[r, pl.ds((j + 1) * _L, _L)]
            part_ap[pl.ds(r * _L, _L)] = ap0 + ap1
            part_an[pl.ds(r * _L, _L)] = an0 + an1
        for g in range(_C // _L):
            sap = _colsum16(part_ap, g)
            san = _colsum16(part_an, g)
            src = pl.ds(c * _C + g * _L, _L)
            ia = plsc.load_gather(invn, [aidx[src]])
            ip = plsc.load_gather(invn, [pidx[src]])
            inn = plsc.load_gather(invn, [nidx[src]])
            d = jnp.float32(2.0) * ia * (san * inn - sap * ip)
            acc = acc + jnp.maximum(d + jnp.float32(_MARGIN), jnp.float32(0.0))
        return acc

    _gather(0, 0, sem0)
    _gather(1, 1, sem1)

    def chunk_pair(t, acc):
        c0 = 2 * t
        more = t < _NCH // 2 - 1

        _gather_wait(c0, 0, sem0)
        acc = _compute(c0, 0, acc)

        @pl.when(more)
        def _():
            _gather(c0 + 2, 0, sem0)

        _gather_wait(c0 + 1, 1, sem1)
        acc = _compute(c0 + 1, 1, acc)

        @pl.when(more)
        def _():
            _gather(c0 + 3, 1, sem1)

        return acc

    acc = lax.fori_loop(0, _NCH // 2, chunk_pair, jnp.zeros((_L,), jnp.float32))
    accv[:] = acc
    pltpu.sync_copy(accv, out_hbm.at[wid])


def kernel(embeddings, triplets):
    emb = embeddings.astype(jnp.float32)
    trip = triplets.astype(jnp.int32)
    ai, pi, ni = trip[:, 0], trip[:, 1], trip[:, 2]

    mesh = plsc.VectorSubcoreMesh(core_axis_name="c", subcore_axis_name="s",
                                  num_cores=_NC, num_subcores=_NS)
    sc = pl.kernel(
        _sc_body,
        out_type=jax.ShapeDtypeStruct((_NW, _L), jnp.float32),
        mesh=mesh,
        compiler_params=pltpu.CompilerParams(needs_layout_passes=False,
                                             use_tc_tiling_on_sc=True),
        scratch_types=[
            pltpu.VMEM((6, _C, _D), jnp.float32),
            pltpu.VMEM((_C * _L,), jnp.float32),
            pltpu.VMEM((_C * _L,), jnp.float32),
            pltpu.VMEM((_V,), jnp.float32),
            pltpu.VMEM((_TPW,), jnp.int32),
            pltpu.VMEM((_TPW,), jnp.int32),
            pltpu.VMEM((_TPW,), jnp.int32),
            pltpu.VMEM((_L,), jnp.float32),
            pltpu.VMEM_SHARED((_V,), jnp.float32),
            pltpu.SemaphoreType.DMA,
            pltpu.SemaphoreType.DMA,
        ],
    )
    partial = sc(emb, ai, pi, ni)
    loss = jnp.sum(partial) / jnp.float32(_T)
    return (loss, triplets.shape[0])
